# baseline (speedup 1.0000x reference)
import functools

import jax
import jax.numpy as jnp
from jax import lax
from jax.experimental import pallas as pl
from jax.experimental.pallas import tpu as pltpu
from jax.experimental.pallas import tpu_sc as plsc

_S = 200
_B = 4096
_D = 64
_C = 512
_SUB = 128
_NSUB = _C // _SUB
_N = _S * _B
_NCHUNK = _N // _C
_CPS = _B // _C
_NC = 2
_NS = 16
_NW = _NC * _NS
_PER_W = _NCHUNK // _NW
_LANES = 16


def _emb_body(x_hbm, table_hbm, pos_hbm, out_hbm, idx_v, rows_v, pos_v, gsem):
    wid = lax.axis_index("s") * _NC + lax.axis_index("c")
    pltpu.sync_copy(pos_hbm.at[pl.ds(0, _S)], pos_v)

    def chunk_body(t, carry):
        g = wid * _PER_W + t
        base = g * _C
        pltpu.sync_copy(x_hbm.at[pl.ds(g * _NSUB, _NSUB)], idx_v)
        copies = [
            pltpu.async_copy(
                table_hbm.at[idx_v.at[j]],
                rows_v.at[pl.ds(j * _SUB, _SUB)],
                gsem,
            )
            for j in range(_NSUB)
        ]
        for cp in copies:
            cp.wait()

        s_idx = g // _CPS
        pos_regs = [
            pos_v[s_idx, pl.ds(d * _LANES, _LANES)] for d in range(_D // _LANES)
        ]

        def row_body(i, c2):
            for d in range(_D // _LANES):
                plsc.addupdate(rows_v.at[i, pl.ds(d * _LANES, _LANES)], pos_regs[d])
            return c2

        lax.fori_loop(0, _C, row_body, 0, unroll=8)
        pltpu.sync_copy(rows_v, out_hbm.at[pl.ds(base, _C)])
        return carry

    lax.fori_loop(0, _PER_W, chunk_body, 0)


@jax.jit
def _run(x, embed_table, pos_table):
    mesh = plsc.VectorSubcoreMesh(core_axis_name="c", subcore_axis_name="s")
    x2d = x.reshape(_N // _SUB, _SUB)
    out = pl.kernel(
        _emb_body,
        out_type=jax.ShapeDtypeStruct((_N, _D), jnp.float32),
        mesh=mesh,
        scratch_types=[
            pltpu.VMEM((_NSUB, _SUB), jnp.int32),
            pltpu.VMEM((_C, _D), jnp.float32),
            pltpu.VMEM((_S, _D), jnp.float32),
            pltpu.SemaphoreType.DMA,
        ],
        compiler_params=pltpu.CompilerParams(use_tc_tiling_on_sc=False),
    )(x2d, embed_table, pos_table)
    return out.reshape(_S, _B, _D)


def kernel(x, embed_table, pos_table):
    return _run(x, embed_table, pos_table)
